# baseline (speedup 1.0000x reference)
import jax
import jax.numpy as jnp
from jax.experimental import pallas as pl

B = 256
M = 16384
NW = 4
NR = 8

B_TILE = 8


def _body(ww_ref, fg_ref, rw_ref, prev_ref, out_ref):
    prev = prev_ref[...]
    p = (1.0 - ww_ref[:, 0 * M : 1 * M]) * (1.0 - ww_ref[:, 1 * M : 2 * M])
    p = p * (1.0 - ww_ref[:, 2 * M : 3 * M]) * (1.0 - ww_ref[:, 3 * M : 4 * M])
    acc = (1.0 - prev) * p
    fg = fg_ref[...]
    free = fg[:, 0:1] * rw_ref[:, 0 * M : 1 * M]
    for r in range(1, NR):
        free = free + fg[:, r : r + 1] * rw_ref[:, r * M : (r + 1) * M]
    out_ref[...] = jnp.clip(1.0 - acc - free, 0.0, 1.0)


def kernel(write_weights, free_gate, read_weights, prev_usage):
    ww2 = write_weights.reshape(B, NW * M)
    rw2 = read_weights.reshape(B, NR * M)
    grid = (B // B_TILE,)
    return pl.pallas_call(
        _body,
        grid=grid,
        in_specs=[
            pl.BlockSpec((B_TILE, NW * M), lambda i: (i, 0)),
            pl.BlockSpec((B_TILE, NR), lambda i: (i, 0)),
            pl.BlockSpec((B_TILE, NR * M), lambda i: (i, 0)),
            pl.BlockSpec((B_TILE, M), lambda i: (i, 0)),
        ],
        out_specs=pl.BlockSpec((B_TILE, M), lambda i: (i, 0)),
        out_shape=jax.ShapeDtypeStruct((B, M), jnp.float32),
    )(ww2, free_gate, rw2, prev_usage)


# speedup vs baseline: 2.1004x; 2.1004x over previous
import functools
import jax
import jax.numpy as jnp
from jax import lax
from jax.experimental import pallas as pl
from jax.experimental.pallas import tpu as pltpu, tpu_sc as plsc

B = 256
M = 16384
NW = 4
NR = 8

B_SC = 96
NWORK = 32
BPW = B_SC // NWORK
CH = 2048
CPB = M // CH
NCH = BPW * CPB
UNROLL = 4
L = 16


def _sc_body(ww_hbm, fg_hbm, rw_hbm, prev_hbm, out_hbm,
             ww_v, rw_v, prev_v, out_v, fg_v,
             in_sem0, in_sem1, out_sem0, out_sem1):
    in_sems = (in_sem0, in_sem1)
    out_sems = (out_sem0, out_sem1)
    wid = lax.axis_index("s") * 2 + lax.axis_index("c")
    b0 = wid * BPW

    pltpu.sync_copy(fg_hbm.at[pl.ds(b0 * NR, BPW * NR)], fg_v.at[pl.ds(0, BPW * NR)])

    def start_in(c, s):
        b = b0 + c // CPB
        m0 = (c % CPB) * CH
        pltpu.async_copy(ww_hbm.at[b, :, pl.ds(m0, CH)], ww_v.at[s], in_sems[s])
        pltpu.async_copy(rw_hbm.at[b, :, pl.ds(m0, CH)], rw_v.at[s], in_sems[s])
        pltpu.async_copy(prev_hbm.at[b, pl.ds(m0, CH)], prev_v.at[s], in_sems[s])

    def wait_in(s):
        pltpu.make_async_copy(ww_hbm.at[0, :, pl.ds(0, CH)], ww_v.at[s], in_sems[s]).wait()
        pltpu.make_async_copy(rw_hbm.at[0, :, pl.ds(0, CH)], rw_v.at[s], in_sems[s]).wait()
        pltpu.make_async_copy(prev_hbm.at[0, pl.ds(0, CH)], prev_v.at[s], in_sems[s]).wait()

    def start_out(c, s):
        b = b0 + c // CPB
        m0 = (c % CPB) * CH
        pltpu.async_copy(out_v.at[s], out_hbm.at[b, pl.ds(m0, CH)], out_sems[s])

    def wait_out(s):
        pltpu.make_async_copy(out_v.at[s], out_hbm.at[0, pl.ds(0, CH)], out_sems[s]).wait()

    start_in(0, 0)
    start_in(1, 1)

    def group_body(g, carry):
        for s in range(2):
            c = 2 * g + s
            wait_in(s)
            bi = c // CPB
            fvec = fg_v[pl.ds(bi * NR, L)]
            fgs = [fvec[r] for r in range(NR)]

            @pl.when(g >= 1)
            def _():
                wait_out(s)

            def vec_body(i, carry2):
                for u in range(UNROLL):
                    sl = pl.ds((i * UNROLL + u) * L, L)
                    p = (1.0 - ww_v[s, 0, sl]) * (1.0 - ww_v[s, 1, sl]) \
                        * (1.0 - ww_v[s, 2, sl]) * (1.0 - ww_v[s, 3, sl])
                    acc = (1.0 - prev_v[s, sl]) * p
                    free = fgs[0] * rw_v[s, 0, sl]
                    for r in range(1, NR):
                        free = free + fgs[r] * rw_v[s, r, sl]
                    res = 1.0 - acc - free
                    out_v[s, sl] = jnp.minimum(jnp.maximum(res, 0.0), 1.0)
                return carry2

            lax.fori_loop(0, CH // (L * UNROLL), vec_body, 0, unroll=False)

            start_out(c, s)

            @pl.when(g < NCH // 2 - 1)
            def _():
                start_in(c + 2, s)
        return carry

    lax.fori_loop(0, NCH // 2, group_body, 0, unroll=False)
    wait_out(0)
    wait_out(1)


def _sc_part(write_weights, free_gate_flat, read_weights, prev_usage):
    mesh = plsc.VectorSubcoreMesh(core_axis_name="c", subcore_axis_name="s")
    f32 = jnp.float32
    k = functools.partial(
        pl.kernel,
        mesh=mesh,
        out_type=jax.ShapeDtypeStruct((B_SC, M), f32),
        scratch_types=[
            pltpu.VMEM((2, NW, CH), f32),
            pltpu.VMEM((2, NR, CH), f32),
            pltpu.VMEM((2, CH), f32),
            pltpu.VMEM((2, CH), f32),
            pltpu.VMEM((BPW * NR + L,), f32),
            pltpu.SemaphoreType.DMA,
            pltpu.SemaphoreType.DMA,
            pltpu.SemaphoreType.DMA,
            pltpu.SemaphoreType.DMA,
        ],
    )(_sc_body)
    return k(write_weights, free_gate_flat, read_weights, prev_usage)


B_TILE = 8
B_TC = B - B_SC
OFF = B_SC // B_TILE


def _tc_body(ww_ref, fg_ref, rw_ref, prev_ref, out_ref):
    prev = prev_ref[...]
    alloc = 1.0 - (
        (1.0 - ww_ref[:, 0, :])
        * (1.0 - ww_ref[:, 1, :])
        * (1.0 - ww_ref[:, 2, :])
        * (1.0 - ww_ref[:, 3, :])
    )
    u = prev + (1.0 - prev) * alloc
    fg = fg_ref[...]
    free = fg[:, 0:1] * rw_ref[:, 0, :]
    for r in range(1, NR):
        free = free + fg[:, r : r + 1] * rw_ref[:, r, :]
    out_ref[...] = jnp.clip(u - free, 0.0, 1.0)


def _tc_part(write_weights, free_gate, read_weights, prev_usage):
    grid = (B_TC // B_TILE,)
    return pl.pallas_call(
        _tc_body,
        grid=grid,
        in_specs=[
            pl.BlockSpec((B_TILE, NW, M), lambda i: (i + OFF, 0, 0)),
            pl.BlockSpec((B_TILE, NR), lambda i: (i + OFF, 0)),
            pl.BlockSpec((B_TILE, NR, M), lambda i: (i + OFF, 0, 0)),
            pl.BlockSpec((B_TILE, M), lambda i: (i + OFF, 0)),
        ],
        out_specs=pl.BlockSpec((B_TILE, M), lambda i: (i, 0)),
        out_shape=jax.ShapeDtypeStruct((B_TC, M), jnp.float32),
    )(write_weights, free_gate, read_weights, prev_usage)


def kernel(write_weights, free_gate, read_weights, prev_usage):
    out_sc = _sc_part(write_weights, free_gate.reshape(B * NR), read_weights, prev_usage)
    out_tc = _tc_part(write_weights, free_gate, read_weights, prev_usage)
    return jnp.concatenate([out_sc, out_tc], axis=0)
